# baseline (speedup 1.0000x reference)
import functools

import jax
import jax.numpy as jnp
from jax import lax
from jax.experimental import pallas as pl
from jax.experimental.pallas import tpu as pltpu
from jax.experimental.pallas import tpu_sc as plsc

N, K, D = 10000, 32, 128
NW = 32
PER_W = 320
N_PAD = NW * PER_W
INV_K = 1.0 / K


def _node_mean(slab, out_v, row):
    for j in range(D // 16):
        s = pl.ds(j * 16, 16)
        vals = [slab[r, s] for r in range(K)]
        while len(vals) > 1:
            vals = [vals[a] + vals[a + 1] for a in range(0, len(vals), 2)]
        out_v[row, s] = vals[0] * INV_K


def _sc_mean_body(nx, out, slab0, slab1, out_v, sem0, sem1):
    w = lax.axis_index("s") * 2 + lax.axis_index("c")
    base = w * PER_W

    def rows(i):
        g = jnp.minimum(base + i, N - 1)
        return pl.ds(g * K, K)

    pltpu.async_copy(nx.at[rows(0)], slab0, sem0)

    def body(t, carry):
        i0 = 2 * t
        i1 = i0 + 1
        pltpu.make_async_copy(nx.at[rows(i0)], slab0, sem0).wait()
        pltpu.async_copy(nx.at[rows(i1)], slab1, sem1)
        _node_mean(slab0, out_v, i0)
        pltpu.make_async_copy(nx.at[rows(i1)], slab1, sem1).wait()
        pltpu.async_copy(nx.at[rows(i0 + 2)], slab0, sem0)
        _node_mean(slab1, out_v, i1)
        return carry

    lax.fori_loop(0, (PER_W + 1) // 2, body, 0)
    pltpu.make_async_copy(nx.at[rows(0)], slab0, sem0).wait()
    pltpu.sync_copy(out_v.at[pl.ds(0, PER_W)], out.at[pl.ds(base, PER_W)])


def _sc_mean(neigh_flat):
    mesh = plsc.VectorSubcoreMesh(core_axis_name="c", subcore_axis_name="s")
    f = functools.partial(
        pl.kernel,
        mesh=mesh,
        out_type=jax.ShapeDtypeStruct((N_PAD, D), jnp.float32),
        scratch_types=[
            pltpu.VMEM((K, D), jnp.float32),
            pltpu.VMEM((K, D), jnp.float32),
            pltpu.VMEM((2 * ((PER_W + 1) // 2), D), jnp.float32),
            pltpu.SemaphoreType.DMA,
            pltpu.SemaphoreType.DMA,
        ],
    )(_sc_mean_body)
    return f(neigh_flat)


def _tc_body(x_ref, m_ref, wl_ref, wr_ref, b_ref, o_ref):
    o_ref[...] = (
        jnp.dot(x_ref[...], wl_ref[...], preferred_element_type=jnp.float32)
        + jnp.dot(m_ref[...], wr_ref[...], preferred_element_type=jnp.float32)
        + b_ref[...]
    )


def kernel(x, neigh_x, W_l, b_l, W_r, b_r):
    n, k, d_in = neigh_x.shape
    d_out = W_l.shape[0]
    mean_pad = _sc_mean(neigh_x.reshape(n * k, d_in))

    block = 1000
    grid = (n // block,)
    out = pl.pallas_call(
        _tc_body,
        grid=grid,
        in_specs=[
            pl.BlockSpec((block, d_in), lambda i: (i, 0)),
            pl.BlockSpec((block, d_in), lambda i: (i, 0)),
            pl.BlockSpec((d_in, d_out), lambda i: (0, 0)),
            pl.BlockSpec((d_in, d_out), lambda i: (0, 0)),
            pl.BlockSpec((1, d_out), lambda i: (0, 0)),
        ],
        out_specs=pl.BlockSpec((block, d_out), lambda i: (i, 0)),
        out_shape=jax.ShapeDtypeStruct((n, d_out), jnp.float32),
        compiler_params=pltpu.CompilerParams(
            dimension_semantics=("arbitrary",),
        ),
    )(x, mean_pad, W_l.T, W_r.T, (b_l + b_r).reshape(1, d_out))
    return out


# speedup vs baseline: 1.2997x; 1.2997x over previous
import functools

import jax
import jax.numpy as jnp
from jax import lax
from jax.experimental import pallas as pl
from jax.experimental.pallas import tpu as pltpu
from jax.experimental.pallas import tpu_sc as plsc

N, K, D = 10000, 32, 128
NW = 32
PER_W = 320
N_PAD = NW * PER_W
INV_K = 1.0 / K
C = 2
RING = 4
CHUNKS = PER_W // C


def _node_mean(slab, u, out_v, row):
    for j in range(D // 16):
        s = pl.ds(j * 16, 16)
        vals = [slab[u * K + r, s] for r in range(K)]
        while len(vals) > 1:
            vals = [vals[a] + vals[a + 1] for a in range(0, len(vals), 2)]
        out_v[row, s] = vals[0] * INV_K


def _sc_mean_body(nx, out, slab0, slab1, slab2, slab3, out_v,
                  sem0, sem1, sem2, sem3):
    slabs = (slab0, slab1, slab2, slab3)
    sems = (sem0, sem1, sem2, sem3)
    w = lax.axis_index("s") * 2 + lax.axis_index("c")
    base = w * PER_W

    def chunk_rows(c):
        g = jnp.minimum(base + C * c, N - C)
        return pl.ds(g * K, C * K)

    for s in range(RING - 1):
        pltpu.async_copy(nx.at[chunk_rows(s)], slabs[s], sems[s])

    def body(t, carry):
        for sub in range(RING):
            c = RING * t + sub
            pltpu.make_async_copy(nx.at[chunk_rows(c)], slabs[sub],
                                  sems[sub]).wait()
            for u in range(C):
                _node_mean(slabs[sub], u, out_v, C * c + u)
            nxt = jnp.minimum(c + RING - 1, CHUNKS - 1)
            nslot = (sub + RING - 1) % RING
            pltpu.async_copy(nx.at[chunk_rows(nxt)], slabs[nslot], sems[nslot])
        return carry

    lax.fori_loop(0, CHUNKS // RING, body, 0)
    for s in range(RING - 1):
        pltpu.make_async_copy(nx.at[chunk_rows(0)], slabs[s], sems[s]).wait()
    pltpu.sync_copy(out_v.at[pl.ds(0, PER_W)], out.at[pl.ds(base, PER_W)])


def _sc_mean(neigh_flat):
    mesh = plsc.VectorSubcoreMesh(core_axis_name="c", subcore_axis_name="s")
    f = functools.partial(
        pl.kernel,
        mesh=mesh,
        out_type=jax.ShapeDtypeStruct((N_PAD, D), jnp.float32),
        scratch_types=[
            pltpu.VMEM((C * K, D), jnp.float32),
            pltpu.VMEM((C * K, D), jnp.float32),
            pltpu.VMEM((C * K, D), jnp.float32),
            pltpu.VMEM((C * K, D), jnp.float32),
            pltpu.VMEM((PER_W, D), jnp.float32),
            pltpu.SemaphoreType.DMA,
            pltpu.SemaphoreType.DMA,
            pltpu.SemaphoreType.DMA,
            pltpu.SemaphoreType.DMA,
        ],
    )(_sc_mean_body)
    return f(neigh_flat)


def _tc_body(x_ref, m_ref, wl_ref, wr_ref, b_ref, o_ref):
    o_ref[...] = (
        jnp.dot(x_ref[...], wl_ref[...], preferred_element_type=jnp.float32)
        + jnp.dot(m_ref[...], wr_ref[...], preferred_element_type=jnp.float32)
        + b_ref[...]
    )


def kernel(x, neigh_x, W_l, b_l, W_r, b_r):
    n, k, d_in = neigh_x.shape
    d_out = W_l.shape[0]
    mean_pad = _sc_mean(neigh_x.reshape(n * k, d_in))

    block = 1000
    grid = (n // block,)
    out = pl.pallas_call(
        _tc_body,
        grid=grid,
        in_specs=[
            pl.BlockSpec((block, d_in), lambda i: (i, 0)),
            pl.BlockSpec((block, d_in), lambda i: (i, 0)),
            pl.BlockSpec((d_in, d_out), lambda i: (0, 0)),
            pl.BlockSpec((d_in, d_out), lambda i: (0, 0)),
            pl.BlockSpec((1, d_out), lambda i: (0, 0)),
        ],
        out_specs=pl.BlockSpec((block, d_out), lambda i: (i, 0)),
        out_shape=jax.ShapeDtypeStruct((n, d_out), jnp.float32),
        compiler_params=pltpu.CompilerParams(
            dimension_semantics=("arbitrary",),
        ),
    )(x, mean_pad, W_l.T, W_r.T, (b_l + b_r).reshape(1, d_out))
    return out


# speedup vs baseline: 2.5296x; 1.9464x over previous
import functools

import jax
import jax.numpy as jnp
from jax import lax
from jax.experimental import pallas as pl
from jax.experimental.pallas import tpu as pltpu
from jax.experimental.pallas import tpu_sc as plsc

N, K, D = 10000, 32, 128
NW = 32
PER_W = 320
N_PAD = NW * PER_W
INV_K = 1.0 / K
C = 2
RING = 4
CHUNKS = PER_W // C


def _node_mean(slab, u, out_v, row):
    for j in range(D // 16):
        s = pl.ds(j * 16, 16)
        vals = [slab[u * K + r, s] for r in range(K)]
        while len(vals) > 1:
            vals = [vals[a] + vals[a + 1] for a in range(0, len(vals), 2)]
        out_v[row, s] = vals[0] * INV_K


def _sc_mean_body(nx, out, slab0, slab1, slab2, slab3, out_v,
                  sem0, sem1, sem2, sem3):
    slabs = (slab0, slab1, slab2, slab3)
    sems = (sem0, sem1, sem2, sem3)
    w = lax.axis_index("s") * 2 + lax.axis_index("c")
    base = w * PER_W

    def chunk_rows(c):
        g = jnp.minimum(base + C * c, N - C)
        return pl.ds(g * K, C * K)

    for s in range(RING - 1):
        pltpu.async_copy(nx.at[chunk_rows(s)], slabs[s], sems[s])

    def body(t, carry):
        for sub in range(RING):
            c = RING * t + sub
            pltpu.make_async_copy(nx.at[chunk_rows(c)], slabs[sub],
                                  sems[sub]).wait()
            nxt = jnp.minimum(c + RING - 1, CHUNKS - 1)
            nslot = (sub + RING - 1) % RING
            pltpu.async_copy(nx.at[chunk_rows(nxt)], slabs[nslot], sems[nslot])
        return carry

    lax.fori_loop(0, CHUNKS // RING, body, 0)
    for s in range(RING - 1):
        pltpu.make_async_copy(nx.at[chunk_rows(0)], slabs[s], sems[s]).wait()
    pltpu.sync_copy(out_v.at[pl.ds(0, PER_W)], out.at[pl.ds(base, PER_W)])


def _sc_mean(neigh_flat):
    mesh = plsc.VectorSubcoreMesh(core_axis_name="c", subcore_axis_name="s")
    f = functools.partial(
        pl.kernel,
        mesh=mesh,
        out_type=jax.ShapeDtypeStruct((N_PAD, D), jnp.float32),
        scratch_types=[
            pltpu.VMEM((C * K, D), jnp.float32),
            pltpu.VMEM((C * K, D), jnp.float32),
            pltpu.VMEM((C * K, D), jnp.float32),
            pltpu.VMEM((C * K, D), jnp.float32),
            pltpu.VMEM((PER_W, D), jnp.float32),
            pltpu.SemaphoreType.DMA,
            pltpu.SemaphoreType.DMA,
            pltpu.SemaphoreType.DMA,
            pltpu.SemaphoreType.DMA,
        ],
    )(_sc_mean_body)
    return f(neigh_flat)


def _tc_body(x_ref, m_ref, wl_ref, wr_ref, b_ref, o_ref):
    o_ref[...] = (
        jnp.dot(x_ref[...], wl_ref[...], preferred_element_type=jnp.float32)
        + jnp.dot(m_ref[...], wr_ref[...], preferred_element_type=jnp.float32)
        + b_ref[...]
    )


def kernel(x, neigh_x, W_l, b_l, W_r, b_r):
    n, k, d_in = neigh_x.shape
    d_out = W_l.shape[0]
    mean_pad = _sc_mean(neigh_x.reshape(n * k, d_in))

    block = 1000
    grid = (n // block,)
    out = pl.pallas_call(
        _tc_body,
        grid=grid,
        in_specs=[
            pl.BlockSpec((block, d_in), lambda i: (i, 0)),
            pl.BlockSpec((block, d_in), lambda i: (i, 0)),
            pl.BlockSpec((d_in, d_out), lambda i: (0, 0)),
            pl.BlockSpec((d_in, d_out), lambda i: (0, 0)),
            pl.BlockSpec((1, d_out), lambda i: (0, 0)),
        ],
        out_specs=pl.BlockSpec((block, d_out), lambda i: (i, 0)),
        out_shape=jax.ShapeDtypeStruct((n, d_out), jnp.float32),
        compiler_params=pltpu.CompilerParams(
            dimension_semantics=("arbitrary",),
        ),
    )(x, mean_pad, W_l.T, W_r.T, (b_l + b_r).reshape(1, d_out))
    return out


# speedup vs baseline: 4.9258x; 1.9472x over previous
import functools

import jax
import jax.numpy as jnp
from jax.experimental import pallas as pl
from jax.experimental.pallas import tpu as pltpu


def _body(x_ref, nx_ref, wl_ref, wr_ref, b_ref, o_ref, *, k):
    mean = jnp.sum(nx_ref[...], axis=1) * (1.0 / k)
    o_ref[...] = (
        jnp.dot(x_ref[...], wl_ref[...], preferred_element_type=jnp.float32)
        + jnp.dot(mean, wr_ref[...], preferred_element_type=jnp.float32)
        + b_ref[...]
    )


def kernel(x, neigh_x, W_l, b_l, W_r, b_r):
    n, k, d_in = neigh_x.shape
    d_out = W_l.shape[0]
    block = 400
    assert n % block == 0
    grid = (n // block,)

    wl_t = W_l.T
    wr_t = W_r.T
    bias = (b_l + b_r).reshape(1, d_out)

    out = pl.pallas_call(
        functools.partial(_body, k=k),
        grid=grid,
        in_specs=[
            pl.BlockSpec((block, d_in), lambda i: (i, 0)),
            pl.BlockSpec((block, k, d_in), lambda i: (i, 0, 0)),
            pl.BlockSpec((d_in, d_out), lambda i: (0, 0)),
            pl.BlockSpec((d_in, d_out), lambda i: (0, 0)),
            pl.BlockSpec((1, d_out), lambda i: (0, 0)),
        ],
        out_specs=pl.BlockSpec((block, d_out), lambda i: (i, 0)),
        out_shape=jax.ShapeDtypeStruct((n, d_out), jnp.float32),
        compiler_params=pltpu.CompilerParams(
            dimension_semantics=("arbitrary",),
        ),
    )(x, neigh_x, wl_t, wr_t, bias)
    return out
